# baseline (speedup 1.0000x reference)
import functools

import jax
import jax.numpy as jnp
from jax import lax
from jax.experimental import pallas as pl
from jax.experimental.pallas import tpu as pltpu
from jax.experimental.pallas import tpu_sc as plsc

B, C, N = 8, 64, 16384
P, S = 2048, 32
NW = 32
WPB = NW // B
CPW = C // WPB
CBLK = 2
NSWEEP = CPW // CBLK
PCH = 128
NCH = P // PCH
T = NSWEEP * NCH
PAIRS_PER_SWEEP = NCH // 2

_mesh = plsc.VectorSubcoreMesh(core_axis_name="c", subcore_axis_name="s")


@functools.partial(
    pl.kernel,
    mesh=_mesh,
    out_type=jax.ShapeDtypeStruct((B, C, S, P), jnp.float32),
    scratch_types=[
        pltpu.VMEM((2 * CBLK, N), jnp.float32),
        pltpu.VMEM((2, S, PCH), jnp.int32),
        pltpu.VMEM((2, CBLK, S, PCH), jnp.float32),
        pltpu.SemaphoreType.DMA((2,)),
        pltpu.SemaphoreType.DMA((2,)),
        pltpu.SemaphoreType.DMA((2,)),
    ],
    compiler_params=pltpu.CompilerParams(needs_layout_passes=False),
)
def _group_sc(feat_hbm, idx_hbm, out_hbm, feat_v, idx_v, out_v,
              isem, osem, fsem):
    cid = lax.axis_index("c")
    sid = lax.axis_index("s")
    w = sid * 2 + cid
    b = w // WPB
    c0 = (w % WPB) * CPW

    def idx_copy(t, buf):
        p0 = lax.rem(t, NCH) * PCH
        return pltpu.make_async_copy(
            idx_hbm.at[b, :, pl.ds(p0, PCH)], idx_v.at[buf], isem.at[buf])

    def out_copy(t, buf):
        cbase = c0 + (t // NCH) * CBLK
        p0 = lax.rem(t, NCH) * PCH
        return pltpu.make_async_copy(
            out_v.at[buf],
            out_hbm.at[b, pl.ds(cbase, CBLK), :, pl.ds(p0, PCH)],
            osem.at[buf])

    def feat_copy(sweep):
        cbase = c0 + sweep * CBLK
        fb = lax.rem(sweep, 2)
        return pltpu.make_async_copy(
            feat_hbm.at[b, pl.ds(cbase, CBLK), :],
            feat_v.at[pl.ds(fb * CBLK, CBLK)], fsem.at[fb])

    def do_chunk(tp, t, buf, frow):
        idx_copy(t, buf).wait()
        @pl.when(t + 1 < T)
        def _():
            idx_copy(t + 1, 1 - buf).start()
        @pl.when(tp > 0)
        def _():
            out_copy(t - 2, buf).wait()

        ccv = [jnp.full((16,), frow + cc, jnp.int32) for cc in range(CBLK)]

        @plsc.parallel_loop(0, (PCH // 16) * S, unroll=8)
        def _gather(i):
            pg = lax.shift_right_logical(i, 5)
            s = lax.bitwise_and(i, S - 1)
            pbase = pg * 16
            iv = idx_v[buf, s, pl.ds(pbase, 16)]
            for cc in range(CBLK):
                out_v[buf, cc, s, pl.ds(pbase, 16)] = plsc.load_gather(
                    feat_v, [ccv[cc], iv])

        out_copy(t, buf).start()

    idx_copy(0, 0).start()
    feat_copy(0).start()

    def pair(tp, _):
        sweep = tp // PAIRS_PER_SWEEP
        @pl.when(lax.rem(tp, PAIRS_PER_SWEEP) == 0)
        def _():
            feat_copy(sweep).wait()
            @pl.when(sweep + 1 < NSWEEP)
            def _():
                feat_copy(sweep + 1).start()

        frow = lax.rem(sweep, 2) * CBLK
        do_chunk(tp, 2 * tp, 0, frow)
        do_chunk(tp, 2 * tp + 1, 1, frow)
        return 0

    lax.fori_loop(0, T // 2, pair, 0)

    out_copy(T - 2, 0).wait()
    out_copy(T - 1, 1).wait()


def kernel(features, idx):
    idx_t = jnp.transpose(idx.astype(jnp.int32), (0, 2, 1))
    out = _group_sc(features, idx_t)
    return jnp.transpose(out, (0, 1, 3, 2))


# speedup vs baseline: 1.2846x; 1.2846x over previous
import functools

import jax
import jax.numpy as jnp
from jax import lax
from jax.experimental import pallas as pl
from jax.experimental.pallas import tpu as pltpu
from jax.experimental.pallas import tpu_sc as plsc

B, C, N = 8, 64, 16384
P, S = 2048, 32
NW = 32
WPB = NW // B
CPW = C // WPB
CBLK = 2
NSWEEP = CPW // CBLK
PCH = 256
NCH = P // PCH
PAIRS = NCH // 2

_mesh = plsc.VectorSubcoreMesh(core_axis_name="c", subcore_axis_name="s")


@functools.partial(
    pl.kernel,
    mesh=_mesh,
    out_type=jax.ShapeDtypeStruct((B, C, S, P), jnp.float32),
    scratch_types=[
        pltpu.VMEM((2 * CBLK, N), jnp.float32),
        pltpu.VMEM((2, S, PCH), jnp.int32),
        pltpu.VMEM((2, CBLK, S, PCH), jnp.float32),
        pltpu.SemaphoreType.DMA((2,)),
        pltpu.SemaphoreType.DMA((2,)),
        pltpu.SemaphoreType.DMA((2,)),
    ],
    compiler_params=pltpu.CompilerParams(needs_layout_passes=False),
)
def _group_sc(feat_hbm, idx_hbm, out_hbm, feat_v, idx_v, out_v,
              isem, osem, fsem):
    cid = lax.axis_index("c")
    sid = lax.axis_index("s")
    w = sid * 2 + cid
    b = w // WPB
    c0 = (w % WPB) * CPW

    def idx_copy(jc, buf):
        p0 = lax.rem(jc, NCH) * PCH
        return pltpu.make_async_copy(
            idx_hbm.at[b, :, pl.ds(p0, PCH)], idx_v.at[buf], isem.at[buf])

    def out_copy(sweep, jc, buf):
        cbase = c0 + sweep * CBLK
        p0 = lax.rem(jc, NCH) * PCH
        return pltpu.make_async_copy(
            out_v.at[buf],
            out_hbm.at[b, pl.ds(cbase, CBLK), :, pl.ds(p0, PCH)],
            osem.at[buf])

    def feat_copy(sweep):
        cbase = c0 + sweep * CBLK
        fb = (sweep % 2) * CBLK
        return pltpu.make_async_copy(
            feat_hbm.at[b, pl.ds(cbase, CBLK), :],
            feat_v.at[pl.ds(fb, CBLK)], fsem.at[sweep % 2])

    ccvs = [jnp.full((16,), r, jnp.int32) for r in range(2 * CBLK)]

    def do_chunk(sweep, jc, buf, first, frow):
        idx_copy(jc, buf).wait()
        idx_copy(jc + 1, 1 - buf).start()
        if first:
            @pl.when(jc > 1)
            def _():
                out_copy(0, jc - 2, buf).wait()
        else:
            out_copy(0, jc - 2, buf).wait()

        @plsc.parallel_loop(0, (PCH // 16) * S, unroll=8)
        def _gather(i):
            pg = lax.shift_right_logical(i, 5)
            s = lax.bitwise_and(i, S - 1)
            pbase = pg * 16
            iv = idx_v[buf, s, pl.ds(pbase, 16)]
            for cc in range(CBLK):
                out_v[buf, cc, s, pl.ds(pbase, 16)] = plsc.load_gather(
                    feat_v, [ccvs[frow + cc], iv])

        out_copy(sweep, jc, buf).start()

    idx_copy(0, 0).start()
    feat_copy(0).start()

    for sweep in range(NSWEEP):
        frow = (sweep % 2) * CBLK
        feat_copy(sweep).wait()
        if sweep + 1 < NSWEEP:
            feat_copy(sweep + 1).start()

        def pair(q, _, sweep=sweep, frow=frow):
            jc = 2 * q
            do_chunk(sweep, jc, 0, sweep == 0, frow)
            do_chunk(sweep, jc + 1, 1, sweep == 0, frow)
            return 0

        lax.fori_loop(0, PAIRS, pair, 0)

    idx_copy(0, 0).wait()
    out_copy(NSWEEP - 1, NCH - 2, 0).wait()
    out_copy(NSWEEP - 1, NCH - 1, 1).wait()


def kernel(features, idx):
    idx_t = jnp.transpose(idx.astype(jnp.int32), (0, 2, 1))
    out = _group_sc(features, idx_t)
    return jnp.transpose(out, (0, 1, 3, 2))


# speedup vs baseline: 1.3122x; 1.0215x over previous
import functools

import jax
import jax.numpy as jnp
from jax import lax
from jax.experimental import pallas as pl
from jax.experimental.pallas import tpu as pltpu
from jax.experimental.pallas import tpu_sc as plsc

B, C, N = 8, 64, 16384
P, S = 2048, 32
NW = 32
WPB = NW // B
CPW = C // WPB
CBLK = 2
NSWEEP = CPW // CBLK
PCH = 256
NCH = P // PCH
T = NSWEEP * NCH
PPS = NCH // 2

_mesh = plsc.VectorSubcoreMesh(core_axis_name="c", subcore_axis_name="s")


@functools.partial(
    pl.kernel,
    mesh=_mesh,
    out_type=jax.ShapeDtypeStruct((B, C, S, P), jnp.float32),
    scratch_types=[
        pltpu.VMEM((2 * CBLK, N), jnp.float32),
        pltpu.VMEM((2, S, PCH), jnp.int32),
        pltpu.VMEM((2, CBLK, S, PCH), jnp.float32),
        pltpu.SemaphoreType.DMA((2,)),
        pltpu.SemaphoreType.DMA((2,)),
        pltpu.SemaphoreType.DMA((2,)),
    ],
    compiler_params=pltpu.CompilerParams(needs_layout_passes=False),
)
def _group_sc(feat_hbm, idx_hbm, out_hbm, feat_v, idx_v, out_v,
              isem, osem, fsem):
    cid = lax.axis_index("c")
    sid = lax.axis_index("s")
    w = sid * 2 + cid
    b = w // WPB
    c0 = (w % WPB) * CPW

    def idx_copy(t, buf):
        p0 = lax.rem(t, NCH) * PCH
        return pltpu.make_async_copy(
            idx_hbm.at[b, :, pl.ds(p0, PCH)], idx_v.at[buf], isem.at[buf])

    def out_copy(t, buf):
        cbase = c0 + (t // NCH) * CBLK
        p0 = lax.rem(t, NCH) * PCH
        return pltpu.make_async_copy(
            out_v.at[buf],
            out_hbm.at[b, pl.ds(cbase, CBLK), :, pl.ds(p0, PCH)],
            osem.at[buf])

    def feat_copy(sweep):
        cbase = c0 + sweep * CBLK
        fb = lax.rem(sweep, 2)
        return pltpu.make_async_copy(
            feat_hbm.at[b, pl.ds(cbase, CBLK), :],
            feat_v.at[pl.ds(fb * CBLK, CBLK)], fsem.at[fb])

    def do_chunk(tp, t, buf, frow):
        idx_copy(t, buf).wait()
        @pl.when(t + 1 < T)
        def _():
            idx_copy(t + 1, 1 - buf).start()
        @pl.when(tp > 0)
        def _():
            out_copy(t - 2, buf).wait()

        def gather_loop(rbase):
            ccv = [jnp.full((16,), rbase + cc, jnp.int32)
                   for cc in range(CBLK)]

            @plsc.parallel_loop(0, (PCH // 16) * S, unroll=8)
            def _gather(i):
                pg = lax.shift_right_logical(i, 5)
                s = lax.bitwise_and(i, S - 1)
                pbase = pg * 16
                iv = idx_v[buf, s, pl.ds(pbase, 16)]
                for cc in range(CBLK):
                    out_v[buf, cc, s, pl.ds(pbase, 16)] = plsc.load_gather(
                        feat_v, [ccv[cc], iv])

        @pl.when(frow == 0)
        def _():
            gather_loop(0)

        @pl.when(frow != 0)
        def _():
            gather_loop(CBLK)

        out_copy(t, buf).start()

    idx_copy(0, 0).start()
    feat_copy(0).start()

    def pair(tp, _):
        sweep = tp // PPS
        @pl.when(lax.rem(tp, PPS) == 0)
        def _():
            feat_copy(sweep).wait()
            @pl.when(sweep + 1 < NSWEEP)
            def _():
                feat_copy(sweep + 1).start()

        frow = lax.rem(sweep, 2) * CBLK
        do_chunk(tp, 2 * tp, 0, frow)
        do_chunk(tp, 2 * tp + 1, 1, frow)
        return 0

    lax.fori_loop(0, T // 2, pair, 0)

    out_copy(T - 2, 0).wait()
    out_copy(T - 1, 1).wait()


def kernel(features, idx):
    idx_t = jnp.transpose(idx.astype(jnp.int32), (0, 2, 1))
    out = _group_sc(features, idx_t)
    return jnp.transpose(out, (0, 1, 3, 2))


# speedup vs baseline: 1.3693x; 1.0436x over previous
import functools

import jax
import jax.numpy as jnp
from jax import lax
from jax.experimental import pallas as pl
from jax.experimental.pallas import tpu as pltpu
from jax.experimental.pallas import tpu_sc as plsc

B, C, N = 8, 64, 16384
P, S = 2048, 32
NW = 32
WPB = NW // B
CPW = C // WPB
CBLK = 4
NSWEEP = CPW // CBLK
PCH = 128
NCH = P // PCH
T = NSWEEP * NCH
OBUF = 3

_mesh = plsc.VectorSubcoreMesh(core_axis_name="c", subcore_axis_name="s")


@functools.partial(
    pl.kernel,
    mesh=_mesh,
    out_type=jax.ShapeDtypeStruct((B, C, S, P), jnp.float32),
    scratch_types=[
        pltpu.VMEM((CBLK, N), jnp.float32),
        pltpu.VMEM((2, S, PCH), jnp.int32),
        pltpu.VMEM((OBUF, CBLK, S, PCH), jnp.float32),
        pltpu.SemaphoreType.DMA((2,)),
        pltpu.SemaphoreType.DMA((OBUF,)),
        pltpu.SemaphoreType.DMA,
    ],
    compiler_params=pltpu.CompilerParams(needs_layout_passes=False),
)
def _group_sc(feat_hbm, idx_hbm, out_hbm, feat_v, idx_v, out_v,
              isem, osem, fsem):
    cid = lax.axis_index("c")
    sid = lax.axis_index("s")
    w = sid * 2 + cid
    b = w // WPB
    c0 = (w % WPB) * CPW

    def idx_copy(t, buf):
        p0 = lax.rem(t, NCH) * PCH
        return pltpu.make_async_copy(
            idx_hbm.at[b, :, pl.ds(p0, PCH)], idx_v.at[buf], isem.at[buf])

    def out_copy(t, buf):
        cbase = c0 + (t // NCH) * CBLK
        p0 = lax.rem(t, NCH) * PCH
        return pltpu.make_async_copy(
            out_v.at[buf],
            out_hbm.at[b, pl.ds(cbase, CBLK), :, pl.ds(p0, PCH)],
            osem.at[buf])

    def feat_copy(sweep):
        cbase = c0 + sweep * CBLK
        return pltpu.make_async_copy(
            feat_hbm.at[b, pl.ds(cbase, CBLK), :], feat_v, fsem)

    ccv = [jnp.full((16,), cc, jnp.int32) for cc in range(CBLK)]

    idx_copy(0, 0).start()

    def chunk(t, _):
        @pl.when(lax.rem(t, NCH) == 0)
        def _():
            fc = feat_copy(t // NCH)
            fc.start()
            fc.wait()

        ib = lax.rem(t, 2)
        ob = lax.rem(t, OBUF)

        idx_copy(t, ib).wait()
        @pl.when(t + 1 < T)
        def _():
            idx_copy(t + 1, 1 - ib).start()
        @pl.when(t >= OBUF)
        def _():
            out_copy(t - OBUF, ob).wait()

        @plsc.parallel_loop(0, (PCH // 16) * S, unroll=8)
        def _gather(i):
            pg = lax.shift_right_logical(i, 5)
            s = lax.bitwise_and(i, S - 1)
            pbase = pg * 16
            iv = idx_v[ib, s, pl.ds(pbase, 16)]
            for cc in range(CBLK):
                out_v[ob, cc, s, pl.ds(pbase, 16)] = plsc.load_gather(
                    feat_v, [ccv[cc], iv])

        out_copy(t, ob).start()
        return 0

    lax.fori_loop(0, T, chunk, 0)

    for k in range(OBUF):
        t = T - OBUF + k
        out_copy(t, lax.rem(jnp.int32(t), OBUF)).wait()


def kernel(features, idx):
    idx_t = jnp.transpose(idx.astype(jnp.int32), (0, 2, 1))
    out = _group_sc(features, idx_t)
    return jnp.transpose(out, (0, 1, 3, 2))


# speedup vs baseline: 1.4261x; 1.0415x over previous
import functools

import jax
import jax.numpy as jnp
from jax import lax
from jax.experimental import pallas as pl
from jax.experimental.pallas import tpu as pltpu
from jax.experimental.pallas import tpu_sc as plsc

B, C, N = 8, 64, 16384
P, S = 2048, 32
NW = 32
WPB = NW // B
CPW = C // WPB
CBLK = 4
NSWEEP = CPW // CBLK
PCH = 128
NCH = P // PCH
T = NSWEEP * NCH
PPS = NCH // 2

_mesh = plsc.VectorSubcoreMesh(core_axis_name="c", subcore_axis_name="s")


@functools.partial(
    pl.kernel,
    mesh=_mesh,
    out_type=jax.ShapeDtypeStruct((B, C, S, P), jnp.float32),
    scratch_types=[
        pltpu.VMEM((CBLK, N), jnp.float32),
        pltpu.VMEM((2, S, PCH), jnp.int32),
        pltpu.VMEM((2, CBLK, S, PCH), jnp.float32),
        pltpu.SemaphoreType.DMA((2,)),
        pltpu.SemaphoreType.DMA((2,)),
        pltpu.SemaphoreType.DMA,
    ],
    compiler_params=pltpu.CompilerParams(needs_layout_passes=False),
)
def _group_sc(feat_hbm, idx_hbm, out_hbm, feat_v, idx_v, out_v,
              isem, osem, fsem):
    cid = lax.axis_index("c")
    sid = lax.axis_index("s")
    w = sid * 2 + cid
    b = w // WPB
    c0 = (w % WPB) * CPW

    def idx_copy(t, buf):
        p0 = lax.rem(t, NCH) * PCH
        return pltpu.make_async_copy(
            idx_hbm.at[b, :, pl.ds(p0, PCH)], idx_v.at[buf], isem.at[buf])

    def out_copy(t, buf):
        cbase = c0 + (t // NCH) * CBLK
        p0 = lax.rem(t, NCH) * PCH
        return pltpu.make_async_copy(
            out_v.at[buf],
            out_hbm.at[b, pl.ds(cbase, CBLK), :, pl.ds(p0, PCH)],
            osem.at[buf])

    def feat_copy(sweep):
        cbase = c0 + sweep * CBLK
        return pltpu.make_async_copy(
            feat_hbm.at[b, pl.ds(cbase, CBLK), :], feat_v, fsem)

    ccv = [jnp.full((16,), cc, jnp.int32) for cc in range(CBLK)]

    def do_chunk(tp, t, buf, feat_boundary):
        idx_copy(t, buf).wait()
        @pl.when(t + 1 < T)
        def _():
            idx_copy(t + 1, 1 - buf).start()
        @pl.when(tp > 0)
        def _():
            out_copy(t - 2, buf).wait()
        if feat_boundary:
            @pl.when(lax.rem(tp, PPS) == 0)
            def _():
                feat_copy(0).wait()

        @plsc.parallel_loop(0, (PCH // 16) * S, unroll=8)
        def _gather(i):
            pg = lax.shift_right_logical(i, 5)
            s = lax.bitwise_and(i, S - 1)
            pbase = pg * 16
            iv = idx_v[buf, s, pl.ds(pbase, 16)]
            for cc in range(CBLK):
                out_v[buf, cc, s, pl.ds(pbase, 16)] = plsc.load_gather(
                    feat_v, [ccv[cc], iv])

        out_copy(t, buf).start()

    idx_copy(0, 0).start()

    def pair(tp, _):
        @pl.when(lax.rem(tp, PPS) == 0)
        def _():
            feat_copy(tp // PPS).start()

        do_chunk(tp, 2 * tp, 0, True)
        do_chunk(tp, 2 * tp + 1, 1, False)
        return 0

    lax.fori_loop(0, T // 2, pair, 0)

    out_copy(T - 2, 0).wait()
    out_copy(T - 1, 1).wait()


def kernel(features, idx):
    idx_t = jnp.transpose(idx.astype(jnp.int32), (0, 2, 1))
    out = _group_sc(features, idx_t)
    return jnp.transpose(out, (0, 1, 3, 2))
